# baseline (speedup 1.0000x reference)
import functools
import math

import jax
import jax.numpy as jnp
from jax import lax
from jax.experimental import pallas as pl
from jax.experimental.pallas import tpu as pltpu
from jax.experimental.pallas import tpu_sc as plsc

_K = 10
_BLK = 2048
_BIGF = 3.0e38
_NEG = float("-inf")


def _topk_body(n_keys, q_ref, k_ref, idx_ref, w_ref, r_ref, ri_ref):
    b = pl.program_id(0)
    nb = pl.num_programs(0)

    @pl.when(b == 0)
    def _init():
        r_ref[...] = jnp.full(r_ref.shape, _NEG, jnp.float32)
        ri_ref[...] = jnp.zeros(ri_ref.shape, jnp.float32)

    q = q_ref[...]
    k = k_ref[...]
    p = lax.dot_general(q + q, k, (((1,), (1,)), ((), ())),
                        preferred_element_type=jnp.float32)
    k2r = lax.dot_general(jnp.ones((1, 16), jnp.float32), k * k,
                          (((1,), (1,)), ((), ())),
                          precision=lax.Precision.HIGHEST,
                          preferred_element_type=jnp.float32)
    rowcol = lax.broadcasted_iota(jnp.int32, (1, _BLK), 1)
    lim = n_keys - b * _BLK
    s = p - (k2r + jnp.where(rowcol < lim, 0.0, _BIGF))

    colf = lax.broadcasted_iota(jnp.int32, s.shape, 1).astype(jnp.float32)
    lanef = lax.broadcasted_iota(jnp.int32, r_ref.shape, 1).astype(
        jnp.float32)
    boff = (b * _BLK).astype(jnp.float32)
    r0 = r_ref[...]
    ri0 = ri_ref[...]
    m0 = jnp.max(s, axis=1, keepdims=True)

    def _cond(carry):
        _, m, r, _ = carry
        return jnp.max(jnp.where(m > r[:, 9:10], 1.0, 0.0)) > 0.0

    def _step(carry):
        s, m, r, ri = carry
        am = jnp.min(jnp.where(s == m, colf, _BIGF), axis=1, keepdims=True)
        s2 = jnp.where(colf == am, _NEG, s)
        m2 = jnp.max(s2, axis=1, keepdims=True)
        pos = jnp.sum(jnp.where(r[:, :_K] >= m, 1.0, 0.0), axis=1,
                      keepdims=True)
        rsh = jnp.concatenate([r[:, :1], r[:, :15]], axis=1)
        rish = jnp.concatenate([ri[:, :1], ri[:, :15]], axis=1)
        r2 = jnp.where(lanef < pos, r,
                       jnp.where(lanef == pos, m, rsh))
        ri2 = jnp.where(lanef < pos, ri,
                        jnp.where(lanef == pos, am + boff, rish))
        return s2, m2, r2, ri2

    _, _, r, ri = lax.while_loop(_cond, _step, (s, m0, r0, ri0))
    r_ref[...] = r
    ri_ref[...] = ri

    @pl.when(b == nb - 1)
    def _finish():
        v = r[:, :_K]
        i = ri[:, :_K].astype(jnp.int32)
        e = jnp.exp(v - v[:, 0:1])
        w = e / jnp.sum(e, axis=1, keepdims=True)
        idx_ref[...] = jnp.concatenate([i] + [i[:, 0:1]] * 6, axis=1)
        w_ref[...] = jnp.concatenate(
            [jnp.broadcast_to(w[:, j:j + 1], (w.shape[0], 16))
             for j in range(_K)], axis=1)


def _tc_topk(queries, keys_padded, n_keys):
    nq = queries.shape[0]
    nb = keys_padded.shape[0] // _BLK
    return pl.pallas_call(
        functools.partial(_topk_body, n_keys),
        grid=(nb,),
        in_specs=[
            pl.BlockSpec((nq, 16), lambda b: (0, 0)),
            pl.BlockSpec((_BLK, 16), lambda b: (b, 0)),
        ],
        out_specs=[
            pl.BlockSpec((nq, 16), lambda b: (0, 0)),
            pl.BlockSpec((nq, 16 * _K), lambda b: (0, 0)),
        ],
        out_shape=[
            jax.ShapeDtypeStruct((nq, 16), jnp.int32),
            jax.ShapeDtypeStruct((nq, 16 * _K), jnp.float32),
        ],
        scratch_shapes=[
            pltpu.VMEM((nq, 16), jnp.float32),
            pltpu.VMEM((nq, 16), jnp.float32),
        ],
        compiler_params=pltpu.CompilerParams(
            dimension_semantics=("arbitrary",)),
    )(queries, keys_padded)


def _sc_aggregate(keys, idx_flat, w_rep):
    nq = w_rep.shape[0]
    info = plsc.get_sparse_core_info()
    nw = info.num_cores * info.num_subcores
    qpw = nq // nw
    mesh = plsc.VectorSubcoreMesh(core_axis_name="c", subcore_axis_name="s")

    @functools.partial(
        pl.kernel,
        mesh=mesh,
        out_type=jax.ShapeDtypeStruct((nq, 16), jnp.float32),
        scratch_types=[
            pltpu.VMEM((qpw * 16,), jnp.int32),
            pltpu.VMEM((qpw * 16, 16), jnp.float32),
            pltpu.VMEM((qpw, 16 * _K), jnp.float32),
            pltpu.VMEM((qpw, 16), jnp.float32),
            pltpu.SemaphoreType.DMA,
        ],
        compiler_params=pltpu.CompilerParams(use_tc_tiling_on_sc=False),
    )
    def body(keys_hbm, idx_hbm, w_hbm, out_hbm, idx_v, rows_v, w_v, out_v,
             sem):
        wid = lax.axis_index("s") * info.num_cores + lax.axis_index("c")
        qbase = wid * qpw
        pltpu.sync_copy(idx_hbm.at[pl.ds(qbase * 16, qpw * 16)], idx_v)
        pltpu.async_copy(keys_hbm.at[idx_v], rows_v, sem).wait()
        pltpu.sync_copy(w_hbm.at[pl.ds(qbase, qpw)], w_v)
        for q in range(qpw):
            acc = rows_v[q * 16] * w_v[q, pl.ds(0, 16)]
            for j in range(1, _K):
                acc = acc + rows_v[q * 16 + j] * w_v[q, pl.ds(j * 16, 16)]
            out_v[q] = acc
        pltpu.sync_copy(out_v, out_hbm.at[pl.ds(qbase, qpw)])

    return body(keys, idx_flat, w_rep)


def kernel(queries, keys):
    n_keys = keys.shape[0]
    nb = math.ceil(n_keys / _BLK)
    keys_padded = jnp.pad(keys, ((0, nb * _BLK - n_keys), (0, 0)))
    idx16, w_rep = _tc_topk(queries, keys_padded, n_keys)
    return _sc_aggregate(keys, idx16.reshape(-1), w_rep)


# speedup vs baseline: 1.3701x; 1.3701x over previous
import functools
import math

import jax
import jax.numpy as jnp
from jax import lax
from jax.experimental import pallas as pl
from jax.experimental.pallas import tpu as pltpu
from jax.experimental.pallas import tpu_sc as plsc

_K = 10
_BLK = 2048
_BIGF = 3.0e38
_NEG = float("-inf")


def _topk_body(n_keys, q_ref, k_ref, idx_ref, w_ref, r_ref, ri_ref, s_ref):
    b = pl.program_id(0)
    nb = pl.num_programs(0)

    @pl.when(b == 0)
    def _init():
        r_ref[...] = jnp.full(r_ref.shape, _NEG, jnp.float32)
        ri_ref[...] = jnp.zeros(ri_ref.shape, jnp.float32)

    q = q_ref[...]
    k = k_ref[...]
    p = lax.dot_general(q + q, k, (((1,), (1,)), ((), ())),
                        preferred_element_type=jnp.float32)
    k2r = lax.dot_general(jnp.ones((1, 16), jnp.float32), k * k,
                          (((1,), (1,)), ((), ())),
                          precision=lax.Precision.HIGHEST,
                          preferred_element_type=jnp.float32)
    rowcol = lax.broadcasted_iota(jnp.int32, (1, _BLK), 1)
    lim = n_keys - b * _BLK
    s = p - (k2r + jnp.where(rowcol < lim, 0.0, _BIGF))

    colf = lax.broadcasted_iota(jnp.int32, s.shape, 1).astype(jnp.float32)
    lanef = lax.broadcasted_iota(jnp.int32, r_ref.shape, 1).astype(
        jnp.float32)
    boff = (b * _BLK).astype(jnp.float32)
    r0 = r_ref[...]
    ri0 = ri_ref[...]
    m0 = jnp.max(s, axis=1, keepdims=True)
    s_ref[...] = s

    def _cond(carry):
        m, r, _ = carry
        return jnp.max(jnp.where(m > r[:, 9:10], 1.0, 0.0)) > 0.0

    def _step(carry):
        m, r, ri = carry
        s = s_ref[...]
        am = jnp.min(jnp.where(s == m, colf, _BIGF), axis=1, keepdims=True)
        s2 = jnp.where(colf == am, _NEG, s)
        s_ref[...] = s2
        m2 = jnp.max(s2, axis=1, keepdims=True)
        pos = jnp.sum(jnp.where(r[:, :_K] >= m, 1.0, 0.0), axis=1,
                      keepdims=True)
        rsh = jnp.concatenate([r[:, :1], r[:, :15]], axis=1)
        rish = jnp.concatenate([ri[:, :1], ri[:, :15]], axis=1)
        r2 = jnp.where(lanef < pos, r,
                       jnp.where(lanef == pos, m, rsh))
        ri2 = jnp.where(lanef < pos, ri,
                        jnp.where(lanef == pos, am + boff, rish))
        return m2, r2, ri2

    _, r, ri = lax.while_loop(_cond, _step, (m0, r0, ri0))
    r_ref[...] = r
    ri_ref[...] = ri

    @pl.when(b == nb - 1)
    def _finish():
        v = r[:, :_K]
        i = ri[:, :_K].astype(jnp.int32)
        e = jnp.exp(v - v[:, 0:1])
        w = e / jnp.sum(e, axis=1, keepdims=True)
        idx_ref[...] = jnp.concatenate([i] + [i[:, 0:1]] * 6, axis=1)
        w_ref[...] = jnp.concatenate(
            [jnp.broadcast_to(w[:, j:j + 1], (w.shape[0], 16))
             for j in range(_K)], axis=1)


def _tc_topk(queries, keys_padded, n_keys):
    nq = queries.shape[0]
    nb = keys_padded.shape[0] // _BLK
    return pl.pallas_call(
        functools.partial(_topk_body, n_keys),
        grid=(nb,),
        in_specs=[
            pl.BlockSpec((nq, 16), lambda b: (0, 0)),
            pl.BlockSpec((_BLK, 16), lambda b: (b, 0)),
        ],
        out_specs=[
            pl.BlockSpec((nq, 16), lambda b: (0, 0)),
            pl.BlockSpec((nq, 16 * _K), lambda b: (0, 0)),
        ],
        out_shape=[
            jax.ShapeDtypeStruct((nq, 16), jnp.int32),
            jax.ShapeDtypeStruct((nq, 16 * _K), jnp.float32),
        ],
        scratch_shapes=[
            pltpu.VMEM((nq, 16), jnp.float32),
            pltpu.VMEM((nq, 16), jnp.float32),
            pltpu.VMEM((nq, _BLK), jnp.float32),
        ],
        compiler_params=pltpu.CompilerParams(
            dimension_semantics=("arbitrary",)),
    )(queries, keys_padded)


def _sc_aggregate(keys, idx_flat, w_rep):
    nq = w_rep.shape[0]
    info = plsc.get_sparse_core_info()
    nw = info.num_cores * info.num_subcores
    qpw = nq // nw
    mesh = plsc.VectorSubcoreMesh(core_axis_name="c", subcore_axis_name="s")

    @functools.partial(
        pl.kernel,
        mesh=mesh,
        out_type=jax.ShapeDtypeStruct((nq, 16), jnp.float32),
        scratch_types=[
            pltpu.VMEM((qpw * 16,), jnp.int32),
            pltpu.VMEM((qpw * 16, 16), jnp.float32),
            pltpu.VMEM((qpw, 16 * _K), jnp.float32),
            pltpu.VMEM((qpw, 16), jnp.float32),
            pltpu.SemaphoreType.DMA,
        ],
        compiler_params=pltpu.CompilerParams(use_tc_tiling_on_sc=False),
    )
    def body(keys_hbm, idx_hbm, w_hbm, out_hbm, idx_v, rows_v, w_v, out_v,
             sem):
        wid = lax.axis_index("s") * info.num_cores + lax.axis_index("c")
        qbase = wid * qpw
        pltpu.sync_copy(idx_hbm.at[pl.ds(qbase * 16, qpw * 16)], idx_v)
        pltpu.async_copy(keys_hbm.at[idx_v], rows_v, sem).wait()
        pltpu.sync_copy(w_hbm.at[pl.ds(qbase, qpw)], w_v)
        for q in range(qpw):
            acc = rows_v[q * 16] * w_v[q, pl.ds(0, 16)]
            for j in range(1, _K):
                acc = acc + rows_v[q * 16 + j] * w_v[q, pl.ds(j * 16, 16)]
            out_v[q] = acc
        pltpu.sync_copy(out_v, out_hbm.at[pl.ds(qbase, qpw)])

    return body(keys, idx_flat, w_rep)


def kernel(queries, keys):
    n_keys = keys.shape[0]
    nb = math.ceil(n_keys / _BLK)
    keys_padded = jnp.pad(keys, ((0, nb * _BLK - n_keys), (0, 0)))
    idx16, w_rep = _tc_topk(queries, keys_padded, n_keys)
    return _sc_aggregate(keys, idx16.reshape(-1), w_rep)


# speedup vs baseline: 1.4488x; 1.0575x over previous
import functools
import math

import jax
import jax.numpy as jnp
from jax import lax
from jax.experimental import pallas as pl
from jax.experimental.pallas import tpu as pltpu
from jax.experimental.pallas import tpu_sc as plsc

_K = 10
_BLK = 2048
_BIGF = 3.0e38
_NEG = float("-inf")


def _topk_body(n_keys, q_ref, k_ref, idx_ref, w_ref, r_ref, ri_ref, s_ref):
    b = pl.program_id(0)
    nb = pl.num_programs(0)

    @pl.when(b == 0)
    def _init():
        r_ref[...] = jnp.full(r_ref.shape, _NEG, jnp.float32)
        ri_ref[...] = jnp.zeros(ri_ref.shape, jnp.float32)

    q = q_ref[...]
    k = k_ref[...]
    p = lax.dot_general(q + q, k, (((1,), (1,)), ((), ())),
                        preferred_element_type=jnp.float32)
    k2r = lax.dot_general(jnp.ones((1, 16), jnp.float32), k * k,
                          (((1,), (1,)), ((), ())),
                          precision=lax.Precision.HIGHEST,
                          preferred_element_type=jnp.float32)
    rowcol = lax.broadcasted_iota(jnp.int32, (1, _BLK), 1)
    lim = n_keys - b * _BLK
    s = p - (k2r + jnp.where(rowcol < lim, 0.0, _BIGF))

    colf = lax.broadcasted_iota(jnp.int32, s.shape, 1).astype(jnp.float32)
    lanef = lax.broadcasted_iota(jnp.int32, r_ref.shape, 1).astype(
        jnp.float32)
    boff = (b * _BLK).astype(jnp.float32)
    r0 = r_ref[...]
    ri0 = ri_ref[...]
    m0 = jnp.max(s, axis=1, keepdims=True)
    s_ref[...] = s

    def _cond(carry):
        m, r, _ = carry
        return jnp.max(jnp.where(m > r[:, 9:10], 1.0, 0.0)) > 0.0

    def _step(carry):
        m, r, ri = carry
        s = s_ref[...]
        am = jnp.min(jnp.where(s == m, colf, _BIGF), axis=1, keepdims=True)
        s2 = jnp.where(colf == am, _NEG, s)
        s_ref[...] = s2
        m2 = jnp.max(s2, axis=1, keepdims=True)
        pos = jnp.sum(jnp.where(r[:, :_K] >= m, 1.0, 0.0), axis=1,
                      keepdims=True)
        rsh = jnp.concatenate([r[:, :1], r[:, :15]], axis=1)
        rish = jnp.concatenate([ri[:, :1], ri[:, :15]], axis=1)
        r2 = jnp.where(lanef < pos, r,
                       jnp.where(lanef == pos, m, rsh))
        ri2 = jnp.where(lanef < pos, ri,
                        jnp.where(lanef == pos, am + boff, rish))
        return m2, r2, ri2

    carry = _step((m0, r0, ri0))
    _, r, ri = lax.while_loop(_cond, lambda c: _step(_step(c)), carry)
    r_ref[...] = r
    ri_ref[...] = ri

    @pl.when(b == nb - 1)
    def _finish():
        v = r[:, :_K]
        i = ri[:, :_K].astype(jnp.int32)
        e = jnp.exp(v - v[:, 0:1])
        w = e / jnp.sum(e, axis=1, keepdims=True)
        idx_ref[...] = jnp.concatenate([i] + [i[:, 0:1]] * 6, axis=1)
        w_ref[...] = jnp.concatenate(
            [jnp.broadcast_to(w[:, j:j + 1], (w.shape[0], 16))
             for j in range(_K)], axis=1)


def _tc_topk(queries, keys_padded, n_keys):
    nq = queries.shape[0]
    nb = keys_padded.shape[0] // _BLK
    return pl.pallas_call(
        functools.partial(_topk_body, n_keys),
        grid=(nb,),
        in_specs=[
            pl.BlockSpec((nq, 16), lambda b: (0, 0)),
            pl.BlockSpec((_BLK, 16), lambda b: (b, 0)),
        ],
        out_specs=[
            pl.BlockSpec((nq, 16), lambda b: (0, 0)),
            pl.BlockSpec((nq, 16 * _K), lambda b: (0, 0)),
        ],
        out_shape=[
            jax.ShapeDtypeStruct((nq, 16), jnp.int32),
            jax.ShapeDtypeStruct((nq, 16 * _K), jnp.float32),
        ],
        scratch_shapes=[
            pltpu.VMEM((nq, 16), jnp.float32),
            pltpu.VMEM((nq, 16), jnp.float32),
            pltpu.VMEM((nq, _BLK), jnp.float32),
        ],
        compiler_params=pltpu.CompilerParams(
            dimension_semantics=("arbitrary",)),
    )(queries, keys_padded)


def _sc_aggregate(keys, idx_flat, w_rep):
    nq = w_rep.shape[0]
    info = plsc.get_sparse_core_info()
    nw = info.num_cores * info.num_subcores
    qpw = nq // nw
    mesh = plsc.VectorSubcoreMesh(core_axis_name="c", subcore_axis_name="s")

    @functools.partial(
        pl.kernel,
        mesh=mesh,
        out_type=jax.ShapeDtypeStruct((nq, 16), jnp.float32),
        scratch_types=[
            pltpu.VMEM((qpw * 16,), jnp.int32),
            pltpu.VMEM((qpw * 16, 16), jnp.float32),
            pltpu.VMEM((qpw, 16 * _K), jnp.float32),
            pltpu.VMEM((qpw, 16), jnp.float32),
            pltpu.SemaphoreType.DMA,
        ],
        compiler_params=pltpu.CompilerParams(use_tc_tiling_on_sc=False),
    )
    def body(keys_hbm, idx_hbm, w_hbm, out_hbm, idx_v, rows_v, w_v, out_v,
             sem):
        wid = lax.axis_index("s") * info.num_cores + lax.axis_index("c")
        qbase = wid * qpw
        pltpu.sync_copy(idx_hbm.at[pl.ds(qbase * 16, qpw * 16)], idx_v)
        pltpu.async_copy(keys_hbm.at[idx_v], rows_v, sem).wait()
        pltpu.sync_copy(w_hbm.at[pl.ds(qbase, qpw)], w_v)
        for q in range(qpw):
            acc = rows_v[q * 16] * w_v[q, pl.ds(0, 16)]
            for j in range(1, _K):
                acc = acc + rows_v[q * 16 + j] * w_v[q, pl.ds(j * 16, 16)]
            out_v[q] = acc
        pltpu.sync_copy(out_v, out_hbm.at[pl.ds(qbase, qpw)])

    return body(keys, idx_flat, w_rep)


def kernel(queries, keys):
    n_keys = keys.shape[0]
    nb = math.ceil(n_keys / _BLK)
    keys_padded = jnp.pad(keys, ((0, nb * _BLK - n_keys), (0, 0)))
    idx16, w_rep = _tc_topk(queries, keys_padded, n_keys)
    return _sc_aggregate(keys, idx16.reshape(-1), w_rep)


# speedup vs baseline: 1.4512x; 1.0017x over previous
import functools
import math

import jax
import jax.numpy as jnp
from jax import lax
from jax.experimental import pallas as pl
from jax.experimental.pallas import tpu as pltpu
from jax.experimental.pallas import tpu_sc as plsc

_K = 10
_BLK = 2048
_BIGF = 3.0e38
_NEG = float("-inf")


def _topk_body(n_keys, q_ref, k_ref, idx_ref, w_ref, r_ref, ri_ref, s_ref):
    b = pl.program_id(0)
    nb = pl.num_programs(0)

    @pl.when(b == 0)
    def _init():
        r_ref[...] = jnp.full(r_ref.shape, _NEG, jnp.float32)
        ri_ref[...] = jnp.zeros(ri_ref.shape, jnp.float32)

    q = q_ref[...]
    k = k_ref[...]
    p = lax.dot_general(q + q, k, (((1,), (1,)), ((), ())),
                        preferred_element_type=jnp.float32)
    k2r = lax.dot_general(jnp.ones((1, 16), jnp.float32), k * k,
                          (((1,), (1,)), ((), ())),
                          precision=lax.Precision.HIGHEST,
                          preferred_element_type=jnp.float32)
    rowcol = lax.broadcasted_iota(jnp.int32, (1, _BLK), 1)
    lim = n_keys - b * _BLK
    s = p - (k2r + jnp.where(rowcol < lim, 0.0, _BIGF))

    nq = s.shape[0]
    hq = nq // 2
    colf = lax.broadcasted_iota(jnp.int32, (hq, _BLK), 1).astype(jnp.float32)
    lanef = lax.broadcasted_iota(jnp.int32, (hq, 16), 1).astype(jnp.float32)
    boff = (b * _BLK).astype(jnp.float32)
    s_ref[...] = s

    def _half_step(lo, m, r, ri):
        sh = s_ref[lo:lo + hq, :]
        am = jnp.min(jnp.where(sh == m, colf, _BIGF), axis=1, keepdims=True)
        s2 = jnp.where(colf == am, _NEG, sh)
        s_ref[lo:lo + hq, :] = s2
        m2 = jnp.max(s2, axis=1, keepdims=True)
        pos = jnp.sum(jnp.where(r[:, :_K] >= m, 1.0, 0.0), axis=1,
                      keepdims=True)
        rsh = jnp.concatenate([r[:, :1], r[:, :15]], axis=1)
        rish = jnp.concatenate([ri[:, :1], ri[:, :15]], axis=1)
        r2 = jnp.where(lanef < pos, r,
                       jnp.where(lanef == pos, m, rsh))
        ri2 = jnp.where(lanef < pos, ri,
                        jnp.where(lanef == pos, am + boff, rish))
        return m2, r2, ri2

    def _cond(carry):
        ma, ra, _, mb, rb, _ = carry
        act_a = jnp.max(jnp.where(ma > ra[:, 9:10], 1.0, 0.0))
        act_b = jnp.max(jnp.where(mb > rb[:, 9:10], 1.0, 0.0))
        return jnp.maximum(act_a, act_b) > 0.0

    def _step(carry):
        ma, ra, ria, mb, rb, rib = carry
        ma2, ra2, ria2 = _half_step(0, ma, ra, ria)
        mb2, rb2, rib2 = _half_step(hq, mb, rb, rib)
        return ma2, ra2, ria2, mb2, rb2, rib2

    m0a = jnp.max(s[:hq], axis=1, keepdims=True)
    m0b = jnp.max(s[hq:], axis=1, keepdims=True)
    carry = (m0a, r_ref[:hq, :], ri_ref[:hq, :],
             m0b, r_ref[hq:, :], ri_ref[hq:, :])
    carry = _step(carry)
    carry = lax.while_loop(_cond, lambda c: _step(_step(c)), carry)
    _, ra, ria, _, rb, rib = carry
    r = jnp.concatenate([ra, rb], axis=0)
    ri = jnp.concatenate([ria, rib], axis=0)
    r_ref[...] = r
    ri_ref[...] = ri

    @pl.when(b == nb - 1)
    def _finish():
        v = r[:, :_K]
        i = ri[:, :_K].astype(jnp.int32)
        e = jnp.exp(v - v[:, 0:1])
        w = e / jnp.sum(e, axis=1, keepdims=True)
        idx_ref[...] = jnp.concatenate([i] + [i[:, 0:1]] * 6, axis=1)
        w_ref[...] = jnp.concatenate(
            [jnp.broadcast_to(w[:, j:j + 1], (w.shape[0], 16))
             for j in range(_K)], axis=1)


def _tc_topk(queries, keys_padded, n_keys):
    nq = queries.shape[0]
    nb = keys_padded.shape[0] // _BLK
    return pl.pallas_call(
        functools.partial(_topk_body, n_keys),
        grid=(nb,),
        in_specs=[
            pl.BlockSpec((nq, 16), lambda b: (0, 0)),
            pl.BlockSpec((_BLK, 16), lambda b: (b, 0)),
        ],
        out_specs=[
            pl.BlockSpec((nq, 16), lambda b: (0, 0)),
            pl.BlockSpec((nq, 16 * _K), lambda b: (0, 0)),
        ],
        out_shape=[
            jax.ShapeDtypeStruct((nq, 16), jnp.int32),
            jax.ShapeDtypeStruct((nq, 16 * _K), jnp.float32),
        ],
        scratch_shapes=[
            pltpu.VMEM((nq, 16), jnp.float32),
            pltpu.VMEM((nq, 16), jnp.float32),
            pltpu.VMEM((nq, _BLK), jnp.float32),
        ],
        compiler_params=pltpu.CompilerParams(
            dimension_semantics=("arbitrary",)),
    )(queries, keys_padded)


def _sc_aggregate(keys, idx_flat, w_rep):
    nq = w_rep.shape[0]
    info = plsc.get_sparse_core_info()
    nw = info.num_cores * info.num_subcores
    qpw = nq // nw
    mesh = plsc.VectorSubcoreMesh(core_axis_name="c", subcore_axis_name="s")

    @functools.partial(
        pl.kernel,
        mesh=mesh,
        out_type=jax.ShapeDtypeStruct((nq, 16), jnp.float32),
        scratch_types=[
            pltpu.VMEM((qpw * 16,), jnp.int32),
            pltpu.VMEM((qpw * 16, 16), jnp.float32),
            pltpu.VMEM((qpw, 16 * _K), jnp.float32),
            pltpu.VMEM((qpw, 16), jnp.float32),
            pltpu.SemaphoreType.DMA,
        ],
        compiler_params=pltpu.CompilerParams(use_tc_tiling_on_sc=False),
    )
    def body(keys_hbm, idx_hbm, w_hbm, out_hbm, idx_v, rows_v, w_v, out_v,
             sem):
        wid = lax.axis_index("s") * info.num_cores + lax.axis_index("c")
        qbase = wid * qpw
        pltpu.sync_copy(idx_hbm.at[pl.ds(qbase * 16, qpw * 16)], idx_v)
        pltpu.async_copy(keys_hbm.at[idx_v], rows_v, sem).wait()
        pltpu.sync_copy(w_hbm.at[pl.ds(qbase, qpw)], w_v)
        for q in range(qpw):
            acc = rows_v[q * 16] * w_v[q, pl.ds(0, 16)]
            for j in range(1, _K):
                acc = acc + rows_v[q * 16 + j] * w_v[q, pl.ds(j * 16, 16)]
            out_v[q] = acc
        pltpu.sync_copy(out_v, out_hbm.at[pl.ds(qbase, qpw)])

    return body(keys, idx_flat, w_rep)


def kernel(queries, keys):
    n_keys = keys.shape[0]
    nb = math.ceil(n_keys / _BLK)
    keys_padded = jnp.pad(keys, ((0, nb * _BLK - n_keys), (0, 0)))
    idx16, w_rep = _tc_topk(queries, keys_padded, n_keys)
    return _sc_aggregate(keys, idx16.reshape(-1), w_rep)


# speedup vs baseline: 1.4541x; 1.0020x over previous
import functools
import math

import jax
import jax.numpy as jnp
from jax import lax
from jax.experimental import pallas as pl
from jax.experimental.pallas import tpu as pltpu
from jax.experimental.pallas import tpu_sc as plsc

_K = 10
_BLK = 2000
_BIGF = 3.0e38
_NEG = float("-inf")


def _topk_body(n_keys, q_ref, k_ref, idx_ref, w_ref, r_ref, ri_ref, s_ref):
    b = pl.program_id(0)
    nb = pl.num_programs(0)

    @pl.when(b == 0)
    def _init():
        r_ref[...] = jnp.full(r_ref.shape, _NEG, jnp.float32)
        ri_ref[...] = jnp.zeros(ri_ref.shape, jnp.float32)

    q = q_ref[...]
    k = k_ref[...]
    p = lax.dot_general(q + q, k, (((1,), (1,)), ((), ())),
                        preferred_element_type=jnp.float32)
    k2r = lax.dot_general(jnp.ones((1, 16), jnp.float32), k * k,
                          (((1,), (1,)), ((), ())),
                          precision=lax.Precision.HIGHEST,
                          preferred_element_type=jnp.float32)
    rowcol = lax.broadcasted_iota(jnp.int32, (1, _BLK), 1)
    lim = n_keys - b * _BLK
    s = p - (k2r + jnp.where(rowcol < lim, 0.0, _BIGF))

    nq = s.shape[0]
    hq = nq // 2
    colf = lax.broadcasted_iota(jnp.int32, (hq, _BLK), 1).astype(jnp.float32)
    lanef = lax.broadcasted_iota(jnp.int32, (hq, 16), 1).astype(jnp.float32)
    boff = (b * _BLK).astype(jnp.float32)
    s_ref[...] = s

    def _half_step(lo, m, r, ri):
        sh = s_ref[lo:lo + hq, :]
        am = jnp.min(jnp.where(sh == m, colf, _BIGF), axis=1, keepdims=True)
        s2 = jnp.where(colf == am, _NEG, sh)
        s_ref[lo:lo + hq, :] = s2
        m2 = jnp.max(s2, axis=1, keepdims=True)
        pos = jnp.sum(jnp.where(r[:, :_K] >= m, 1.0, 0.0), axis=1,
                      keepdims=True)
        rsh = jnp.concatenate([r[:, :1], r[:, :15]], axis=1)
        rish = jnp.concatenate([ri[:, :1], ri[:, :15]], axis=1)
        r2 = jnp.where(lanef < pos, r,
                       jnp.where(lanef == pos, m, rsh))
        ri2 = jnp.where(lanef < pos, ri,
                        jnp.where(lanef == pos, am + boff, rish))
        return m2, r2, ri2

    def _cond(carry):
        return carry[-1] > 0.0

    def _step(carry):
        ma, ra, ria, mb, rb, rib, _ = carry
        ma2, ra2, ria2 = _half_step(0, ma, ra, ria)
        mb2, rb2, rib2 = _half_step(hq, mb, rb, rib)
        act = jnp.maximum(
            jnp.max(jnp.where(ma2 > ra2[:, 9:10], 1.0, 0.0)),
            jnp.max(jnp.where(mb2 > rb2[:, 9:10], 1.0, 0.0)))
        return ma2, ra2, ria2, mb2, rb2, rib2, act

    m0a = jnp.max(s[:hq], axis=1, keepdims=True)
    m0b = jnp.max(s[hq:], axis=1, keepdims=True)
    carry = (m0a, r_ref[:hq, :], ri_ref[:hq, :],
             m0b, r_ref[hq:, :], ri_ref[hq:, :], jnp.max(m0a))
    carry = _step(carry)
    carry = lax.while_loop(_cond, lambda c: _step(_step(c)), carry)
    _, ra, ria, _, rb, rib, _ = carry
    r = jnp.concatenate([ra, rb], axis=0)
    ri = jnp.concatenate([ria, rib], axis=0)
    r_ref[...] = r
    ri_ref[...] = ri

    @pl.when(b == nb - 1)
    def _finish():
        v = r[:, :_K]
        i = ri[:, :_K].astype(jnp.int32)
        e = jnp.exp(v - v[:, 0:1])
        w = e / jnp.sum(e, axis=1, keepdims=True)
        idx_ref[...] = jnp.concatenate([i] + [i[:, 0:1]] * 6, axis=1)
        w_ref[...] = jnp.concatenate(
            [jnp.broadcast_to(w[:, j:j + 1], (w.shape[0], 16))
             for j in range(_K)], axis=1)


def _tc_topk(queries, keys_padded, n_keys):
    nq = queries.shape[0]
    nb = keys_padded.shape[0] // _BLK
    return pl.pallas_call(
        functools.partial(_topk_body, n_keys),
        grid=(nb,),
        in_specs=[
            pl.BlockSpec((nq, 16), lambda b: (0, 0)),
            pl.BlockSpec((_BLK, 16), lambda b: (b, 0)),
        ],
        out_specs=[
            pl.BlockSpec((nq, 16), lambda b: (0, 0)),
            pl.BlockSpec((nq, 16 * _K), lambda b: (0, 0)),
        ],
        out_shape=[
            jax.ShapeDtypeStruct((nq, 16), jnp.int32),
            jax.ShapeDtypeStruct((nq, 16 * _K), jnp.float32),
        ],
        scratch_shapes=[
            pltpu.VMEM((nq, 16), jnp.float32),
            pltpu.VMEM((nq, 16), jnp.float32),
            pltpu.VMEM((nq, _BLK), jnp.float32),
        ],
        compiler_params=pltpu.CompilerParams(
            dimension_semantics=("arbitrary",)),
    )(queries, keys_padded)


def _sc_aggregate(keys, idx_flat, w_rep):
    nq = w_rep.shape[0]
    info = plsc.get_sparse_core_info()
    nw = info.num_cores * info.num_subcores
    qpw = nq // nw
    mesh = plsc.VectorSubcoreMesh(core_axis_name="c", subcore_axis_name="s")

    @functools.partial(
        pl.kernel,
        mesh=mesh,
        out_type=jax.ShapeDtypeStruct((nq, 16), jnp.float32),
        scratch_types=[
            pltpu.VMEM((qpw * 16,), jnp.int32),
            pltpu.VMEM((qpw * 16, 16), jnp.float32),
            pltpu.VMEM((qpw, 16 * _K), jnp.float32),
            pltpu.VMEM((qpw, 16), jnp.float32),
            pltpu.SemaphoreType.DMA,
        ],
        compiler_params=pltpu.CompilerParams(use_tc_tiling_on_sc=False),
    )
    def body(keys_hbm, idx_hbm, w_hbm, out_hbm, idx_v, rows_v, w_v, out_v,
             sem):
        wid = lax.axis_index("s") * info.num_cores + lax.axis_index("c")
        qbase = wid * qpw
        pltpu.sync_copy(idx_hbm.at[pl.ds(qbase * 16, qpw * 16)], idx_v)
        pltpu.async_copy(keys_hbm.at[idx_v], rows_v, sem).wait()
        pltpu.sync_copy(w_hbm.at[pl.ds(qbase, qpw)], w_v)
        for q in range(qpw):
            acc = rows_v[q * 16] * w_v[q, pl.ds(0, 16)]
            for j in range(1, _K):
                acc = acc + rows_v[q * 16 + j] * w_v[q, pl.ds(j * 16, 16)]
            out_v[q] = acc
        pltpu.sync_copy(out_v, out_hbm.at[pl.ds(qbase, qpw)])

    return body(keys, idx_flat, w_rep)


def kernel(queries, keys):
    n_keys = keys.shape[0]
    nb = math.ceil(n_keys / _BLK)
    keys_padded = jnp.pad(keys, ((0, nb * _BLK - n_keys), (0, 0)))
    idx16, w_rep = _tc_topk(queries, keys_padded, n_keys)
    return _sc_aggregate(keys, idx16.reshape(-1), w_rep)
